# baseline (speedup 1.0000x reference)
import functools

import jax
import jax.numpy as jnp
from jax import lax
from jax.experimental import pallas as pl
from jax.experimental.pallas import tpu as pltpu
from jax.experimental.pallas import tpu_sc as plsc

N_NODES = 10000
N_EDGES = 320000
D_IN = 128
D_HID = 128
D_OUT = 40

NC = 2
NS = 16
NW = NC * NS
CHUNK = 128
NCHUNK_TOT = N_EDGES // CHUNK
NCHUNK_LO = NCHUNK_TOT // NW
NEXTRA = NCHUNK_TOT - NW * NCHUNK_LO
N_PAD = 10240
ROWS_PER_TILE = N_PAD // NS


@functools.lru_cache(maxsize=None)
def _make_agg(with_counts):
    D = D_HID
    mesh = plsc.VectorSubcoreMesh(core_axis_name="c", subcore_axis_name="s")

    out_type = [jax.ShapeDtypeStruct((NC, N_PAD, D), jnp.float32)]
    scratch = [
        pltpu.VMEM_SHARED((N_PAD, D), jnp.float32),
        pltpu.VMEM((3, CHUNK), jnp.int32),
        pltpu.VMEM((3, CHUNK), jnp.int32),
        pltpu.VMEM((2 * CHUNK, D), jnp.float32),
        pltpu.SemaphoreType.DMA,
        pltpu.SemaphoreType.DMA((3,)),
    ]
    if with_counts:
        out_type.append(jax.ShapeDtypeStruct((NC, N_PAD), jnp.float32))
        scratch += [
            pltpu.VMEM((N_PAD,), jnp.float32),
            pltpu.VMEM_SHARED((2, N_PAD), jnp.float32),
            pltpu.VMEM((2, ROWS_PER_TILE), jnp.float32),
            pltpu.VMEM((ROWS_PER_TILE,), jnp.float32),
        ]

    @functools.partial(
        pl.kernel,
        out_type=out_type,
        mesh=mesh,
        compiler_params=pltpu.CompilerParams(needs_layout_passes=False),
        scratch_types=scratch,
    )
    def agg(x_hbm, edges_hbm, zeros_hbm, *refs):
        if with_counts:
            (out_hbm, cnt_hbm, acc_sh, src_v, dst_v, rows_v, sem_g, sem_i,
             hist_v, hist_sh, merge_v, csum_v) = refs
        else:
            out_hbm, acc_sh, src_v, dst_v, rows_v, sem_g, sem_i = refs
        c = lax.axis_index("c")
        s = lax.axis_index("s")
        pltpu.sync_copy(zeros_hbm, acc_sh.at[pl.ds(s * ROWS_PER_TILE, ROWS_PER_TILE)])
        wid = c * NS + s
        ncw = NCHUNK_LO + jnp.where(wid < NEXTRA, 1, 0)
        base = (wid * NCHUNK_LO + jnp.minimum(wid, NEXTRA)) * CHUNK
        if with_counts:
            def zbody(i, carry):
                hist_v[pl.ds(i * 16, 16)] = jnp.zeros((16,), jnp.float32)
                return carry
            lax.fori_loop(0, N_PAD // 16, zbody, 0)
        plsc.subcore_barrier()

        def buf(b):
            return rows_v.at[pl.ds(b * CHUNK, CHUNK)]

        def idx_load(j, r):
            off = base + j * CHUNK
            pltpu.async_copy(edges_hbm.at[pl.ds(off, CHUNK)], src_v.at[r],
                             sem_i.at[r])
            pltpu.async_copy(edges_hbm.at[pl.ds(N_EDGES + off, CHUNK)],
                             dst_v.at[r], sem_i.at[r])

        def idx_wait(r):
            pltpu.make_async_copy(edges_hbm.at[pl.ds(0, CHUNK)], src_v.at[0],
                                  sem_i.at[r]).wait()
            pltpu.make_async_copy(edges_hbm.at[pl.ds(0, CHUNK)], dst_v.at[0],
                                  sem_i.at[r]).wait()

        def gather(i, b, r):
            pltpu.async_copy(x_hbm.at[src_v.at[r]], buf(b), sem_g)

        def scatter(i, b, r):
            pltpu.sync_copy(buf(b), acc_sh.at[dst_v.at[r]], add=True)
            if with_counts:
                for j in range(CHUNK // 16):
                    d16 = dst_v[r, pl.ds(j * 16, 16)]
                    c16, last = plsc.scan_count(d16)
                    plsc.addupdate_scatter(
                        hist_v, [d16], c16.astype(jnp.float32), mask=last)

        def drain_g():
            pltpu.make_async_copy(x_hbm.at[src_v.at[0]], buf(0), sem_g).wait()

        idx_load(0, 0)
        idx_load(1, 1)
        idx_wait(0)
        gather(0, 0, 0)

        def body(i, carry):
            b = lax.rem(i, 2)
            r = lax.rem(i, 3)
            drain_g()

            @pl.when(i + 2 <= ncw - 1)
            def _():
                idx_load(i + 2, lax.rem(i + 2, 3))

            @pl.when(i + 1 <= ncw - 1)
            def _():
                r1 = lax.rem(i + 1, 3)
                idx_wait(r1)
                gather(i + 1, 1 - b, r1)

            scatter(i, b, r)
            return carry

        lax.fori_loop(0, ncw, body, 0)
        plsc.subcore_barrier()
        pltpu.sync_copy(acc_sh.at[pl.ds(s * ROWS_PER_TILE, ROWS_PER_TILE)],
                        out_hbm.at[c, pl.ds(s * ROWS_PER_TILE, ROWS_PER_TILE)])
        if with_counts:
            for r in range(8):
                @pl.when(lax.div(s, 2) == r)
                def _():
                    pltpu.sync_copy(hist_v, hist_sh.at[lax.rem(s, 2)])
                plsc.subcore_barrier()
                pltpu.sync_copy(
                    hist_sh.at[:, pl.ds(s * ROWS_PER_TILE, ROWS_PER_TILE)],
                    merge_v)

                def mbody(g, carry):
                    acc = jnp.zeros((16,), jnp.float32)
                    for rr in range(2):
                        acc = acc + merge_v[rr, pl.ds(g * 16, 16)]
                    if r:
                        acc = acc + csum_v[pl.ds(g * 16, 16)]
                    csum_v[pl.ds(g * 16, 16)] = acc
                    return carry

                lax.fori_loop(0, ROWS_PER_TILE // 16, mbody, 0)
                plsc.subcore_barrier()
            pltpu.sync_copy(
                csum_v,
                cnt_hbm.at[c, pl.ds(s * ROWS_PER_TILE, ROWS_PER_TILE)])

    return agg


def _dot_t(a, w):
    return lax.dot_general(a, w, (((1,), (1,)), ((), ())),
                           preferred_element_type=jnp.float32)


def _dense0_body(p_ref, cnt_ref, x_ref, wl0_ref, bl0_ref, wr0_ref,
                 h0_ref, inv_ref):
    s = p_ref[0] + p_ref[1]
    inv = 1.0 / jnp.maximum(cnt_ref[0] + cnt_ref[1], 1.0)
    mean = s * inv
    h0 = (_dot_t(mean, wl0_ref[...]) + bl0_ref[...]
          + _dot_t(x_ref[...], wr0_ref[...]))
    h0_ref[...] = h0
    inv_ref[...] = inv


def _dense1_body(p_ref, inv_ref, h0_ref, wl1_ref, bl1_ref, wr1_ref,
                 w1_ref, b1_ref, w2_ref, b2_ref, out_ref):
    s = p_ref[0] + p_ref[1]
    mean = s * inv_ref[...]
    h1 = (_dot_t(mean, wl1_ref[...]) + bl1_ref[...]
          + _dot_t(h0_ref[...], wr1_ref[...]))
    z = (_dot_t(h0_ref[...], w1_ref[:, :D_HID])
         + _dot_t(h1, w1_ref[:, D_HID:])
         + b1_ref[...])
    z = jnp.maximum(z, 0.0)
    o = _dot_t(z, w2_ref[...]) + b2_ref[...]
    m = jnp.max(o, axis=1, keepdims=True)
    e = jnp.exp(o - m)
    lse = jnp.log(jnp.sum(e, axis=1, keepdims=True))
    out_ref[...] = o - m - lse


_BLK = 1000
_GRID = N_NODES // _BLK


def _row_spec(d):
    return pl.BlockSpec((_BLK, d), lambda i: (i, 0))


def _part_spec(d):
    return pl.BlockSpec((NC, _BLK, d), lambda i: (0, i, 0))


def _full_spec(r, c):
    return pl.BlockSpec((r, c), lambda i: (0, 0))


def _dense0(p0, cnt_col, x, wl0, bl0r, wr0):
    return pl.pallas_call(
        _dense0_body,
        grid=(_GRID,),
        in_specs=[
            _part_spec(D_IN),
            _part_spec(1),
            _row_spec(D_IN),
            _full_spec(D_HID, D_IN),
            _full_spec(1, D_HID),
            _full_spec(D_HID, D_IN),
        ],
        out_specs=[_row_spec(D_HID), _row_spec(1)],
        out_shape=[
            jax.ShapeDtypeStruct((N_NODES, D_HID), jnp.float32),
            jax.ShapeDtypeStruct((N_NODES, 1), jnp.float32),
        ],
    )(p0, cnt_col, x, wl0, bl0r, wr0)


def _dense1(p1, inv, h0, wl1, bl1r, wr1, w1, b1r, w2, b2r):
    return pl.pallas_call(
        _dense1_body,
        grid=(_GRID,),
        in_specs=[
            _part_spec(D_HID),
            _row_spec(1),
            _row_spec(D_HID),
            _full_spec(D_HID, D_HID),
            _full_spec(1, D_HID),
            _full_spec(D_HID, D_HID),
            _full_spec(D_HID, 2 * D_HID),
            _full_spec(1, D_HID),
            _full_spec(D_OUT, D_HID),
            _full_spec(1, D_OUT),
        ],
        out_specs=pl.BlockSpec((_BLK, D_OUT), lambda i: (i, 0)),
        out_shape=jax.ShapeDtypeStruct((N_NODES, D_OUT), jnp.float32),
    )(p1, inv, h0, wl1, bl1r, wr1, w1, b1r, w2, b2r)


def kernel(x, edge_index, Wl0, bl0, Wr0, Wl1, bl1, Wr1, W1, b1, W2, b2):
    edges = edge_index.astype(jnp.int32).reshape(2 * N_EDGES)
    z_hid = jnp.zeros((ROWS_PER_TILE, D_HID), jnp.float32)

    p0, cnt = _make_agg(True)(x, edges, z_hid)
    h0, inv = _dense0(p0, cnt.reshape(NC, N_PAD, 1), x,
                      Wl0, bl0.reshape(1, -1), Wr0)
    p1 = _make_agg(False)(h0, edges, z_hid)[0]
    return _dense1(p1, inv, h0, Wl1, bl1.reshape(1, -1), Wr1,
                   W1, b1.reshape(1, -1), W2, b2.reshape(1, -1))


# speedup vs baseline: 1.0316x; 1.0316x over previous
import functools

import jax
import jax.numpy as jnp
from jax import lax
from jax.experimental import pallas as pl
from jax.experimental.pallas import tpu as pltpu
from jax.experimental.pallas import tpu_sc as plsc

N_NODES = 10000
N_EDGES = 320000
D_IN = 128
D_HID = 128
D_OUT = 40

NC = 2
NS = 16
NW = NC * NS
CHUNK = 128
NCHUNK_TOT = N_EDGES // CHUNK
NCHUNK_LO = NCHUNK_TOT // NW
NEXTRA = NCHUNK_TOT - NW * NCHUNK_LO
N_PAD = 10240
ROWS_PER_TILE = N_PAD // NS


@functools.lru_cache(maxsize=None)
def _make_agg(with_counts):
    D = D_HID
    mesh = plsc.VectorSubcoreMesh(core_axis_name="c", subcore_axis_name="s")

    out_type = [jax.ShapeDtypeStruct((NC, N_PAD, D), jnp.float32)]
    scratch = [
        pltpu.VMEM_SHARED((N_PAD, D), jnp.float32),
        pltpu.VMEM((3, CHUNK), jnp.int32),
        pltpu.VMEM((3, CHUNK), jnp.int32),
        pltpu.VMEM((2 * CHUNK, D), jnp.float32),
        pltpu.SemaphoreType.DMA,
        pltpu.SemaphoreType.DMA((3,)),
    ]
    if with_counts:
        out_type.append(jax.ShapeDtypeStruct((NC, N_PAD), jnp.float32))
        scratch += [
            pltpu.VMEM((N_PAD,), jnp.float32),
            pltpu.VMEM_SHARED((2, N_PAD), jnp.float32),
            pltpu.VMEM((2, ROWS_PER_TILE), jnp.float32),
            pltpu.VMEM((ROWS_PER_TILE,), jnp.float32),
        ]

    @functools.partial(
        pl.kernel,
        out_type=out_type,
        mesh=mesh,
        compiler_params=pltpu.CompilerParams(needs_layout_passes=False),
        scratch_types=scratch,
    )
    def agg(x_hbm, edges_hbm, zeros_hbm, *refs):
        if with_counts:
            (out_hbm, cnt_hbm, acc_sh, src_v, dst_v, rows_v, sem_g, sem_i,
             hist_v, hist_sh, merge_v, csum_v) = refs
        else:
            out_hbm, acc_sh, src_v, dst_v, rows_v, sem_g, sem_i = refs
        c = lax.axis_index("c")
        s = lax.axis_index("s")
        pltpu.sync_copy(zeros_hbm, acc_sh.at[pl.ds(s * ROWS_PER_TILE, ROWS_PER_TILE)])
        wid = c * NS + s
        ncw = NCHUNK_LO + jnp.where(wid < NEXTRA, 1, 0)
        base = (wid * NCHUNK_LO + jnp.minimum(wid, NEXTRA)) * CHUNK
        if with_counts:
            def zbody(i, carry):
                hist_v[pl.ds(i * 16, 16)] = jnp.zeros((16,), jnp.float32)
                return carry
            lax.fori_loop(0, N_PAD // 16, zbody, 0)
        plsc.subcore_barrier()

        def buf(b):
            return rows_v.at[pl.ds(b * CHUNK, CHUNK)]

        def idx_load(j, r):
            off = base + j * CHUNK
            pltpu.async_copy(edges_hbm.at[pl.ds(off, CHUNK)], src_v.at[r],
                             sem_i.at[r])
            pltpu.async_copy(edges_hbm.at[pl.ds(N_EDGES + off, CHUNK)],
                             dst_v.at[r], sem_i.at[r])

        def idx_wait(r):
            pltpu.make_async_copy(edges_hbm.at[pl.ds(0, CHUNK)], src_v.at[0],
                                  sem_i.at[r]).wait()
            pltpu.make_async_copy(edges_hbm.at[pl.ds(0, CHUNK)], dst_v.at[0],
                                  sem_i.at[r]).wait()

        def gather(i, b, r):
            pltpu.async_copy(x_hbm.at[src_v.at[r]], buf(b), sem_g)

        def scatter(i, b, r):
            pltpu.sync_copy(buf(b), acc_sh.at[dst_v.at[r]], add=True)
            if with_counts:
                for j in range(CHUNK // 16):
                    d16 = dst_v[r, pl.ds(j * 16, 16)]
                    c16, last = plsc.scan_count(d16)
                    plsc.addupdate_scatter(
                        hist_v, [d16], c16.astype(jnp.float32), mask=last)

        def drain_g():
            pltpu.make_async_copy(x_hbm.at[src_v.at[0]], buf(0), sem_g).wait()

        idx_load(0, 0)
        idx_load(1, 1)
        idx_wait(0)
        gather(0, 0, 0)

        def body(i, carry):
            b = lax.rem(i, 2)
            r = lax.rem(i, 3)
            drain_g()

            @pl.when(i + 2 <= ncw - 1)
            def _():
                idx_load(i + 2, lax.rem(i + 2, 3))

            @pl.when(i + 1 <= ncw - 1)
            def _():
                r1 = lax.rem(i + 1, 3)
                idx_wait(r1)
                gather(i + 1, 1 - b, r1)

            scatter(i, b, r)
            return carry

        lax.fori_loop(0, ncw, body, 0)
        plsc.subcore_barrier()
        pltpu.sync_copy(acc_sh.at[pl.ds(s * ROWS_PER_TILE, ROWS_PER_TILE)],
                        out_hbm.at[c, pl.ds(s * ROWS_PER_TILE, ROWS_PER_TILE)])
        if with_counts:
            for r in range(8):
                @pl.when(lax.div(s, 2) == r)
                def _():
                    pltpu.sync_copy(hist_v, hist_sh.at[lax.rem(s, 2)])
                plsc.subcore_barrier()
                pltpu.sync_copy(
                    hist_sh.at[:, pl.ds(s * ROWS_PER_TILE, ROWS_PER_TILE)],
                    merge_v)

                def mbody(g, carry):
                    acc = jnp.zeros((16,), jnp.float32)
                    for rr in range(2):
                        acc = acc + merge_v[rr, pl.ds(g * 16, 16)]
                    if r:
                        acc = acc + csum_v[pl.ds(g * 16, 16)]
                    csum_v[pl.ds(g * 16, 16)] = acc
                    return carry

                lax.fori_loop(0, ROWS_PER_TILE // 16, mbody, 0)
                plsc.subcore_barrier()
            pltpu.sync_copy(
                csum_v,
                cnt_hbm.at[c, pl.ds(s * ROWS_PER_TILE, ROWS_PER_TILE)])

    return agg


def _dot_t(a, w):
    return lax.dot_general(a, w, (((1,), (1,)), ((), ())),
                           preferred_element_type=jnp.float32)


def _dense0_body(p_ref, cnt_ref, x_ref, wl0_ref, bl0_ref, wr0_ref,
                 h0_ref, inv_ref):
    s = p_ref[0] + p_ref[1]
    inv = 1.0 / jnp.maximum(cnt_ref[...], 1.0)
    mean = s * inv
    h0 = (_dot_t(mean, wl0_ref[...]) + bl0_ref[...]
          + _dot_t(x_ref[...], wr0_ref[...]))
    h0_ref[...] = h0
    inv_ref[...] = inv


def _dense1_body(p_ref, inv_ref, h0_ref, wl1_ref, bl1_ref, wr1_ref,
                 w1_ref, b1_ref, w2_ref, b2_ref, out_ref):
    s = p_ref[0] + p_ref[1]
    mean = s * inv_ref[...]
    h1 = (_dot_t(mean, wl1_ref[...]) + bl1_ref[...]
          + _dot_t(h0_ref[...], wr1_ref[...]))
    z = (_dot_t(h0_ref[...], w1_ref[:, :D_HID])
         + _dot_t(h1, w1_ref[:, D_HID:])
         + b1_ref[...])
    z = jnp.maximum(z, 0.0)
    o = _dot_t(z, w2_ref[...]) + b2_ref[...]
    m = jnp.max(o, axis=1, keepdims=True)
    e = jnp.exp(o - m)
    lse = jnp.log(jnp.sum(e, axis=1, keepdims=True))
    out_ref[...] = o - m - lse


_BLK = 2000
_GRID = N_NODES // _BLK


def _row_spec(d):
    return pl.BlockSpec((_BLK, d), lambda i: (i, 0))


def _part_spec(d):
    return pl.BlockSpec((NC, _BLK, d), lambda i: (0, i, 0))


def _full_spec(r, c):
    return pl.BlockSpec((r, c), lambda i: (0, 0))


def _dense0(p0, cnt_col, x, wl0, bl0r, wr0):
    return pl.pallas_call(
        _dense0_body,
        grid=(_GRID,),
        in_specs=[
            _part_spec(D_IN),
            _row_spec(1),
            _row_spec(D_IN),
            _full_spec(D_HID, D_IN),
            _full_spec(1, D_HID),
            _full_spec(D_HID, D_IN),
        ],
        out_specs=[_row_spec(D_HID), _row_spec(1)],
        out_shape=[
            jax.ShapeDtypeStruct((N_NODES, D_HID), jnp.float32),
            jax.ShapeDtypeStruct((N_NODES, 1), jnp.float32),
        ],
    )(p0, cnt_col, x, wl0, bl0r, wr0)


def _dense1(p1, inv, h0, wl1, bl1r, wr1, w1, b1r, w2, b2r):
    return pl.pallas_call(
        _dense1_body,
        grid=(_GRID,),
        in_specs=[
            _part_spec(D_HID),
            _row_spec(1),
            _row_spec(D_HID),
            _full_spec(D_HID, D_HID),
            _full_spec(1, D_HID),
            _full_spec(D_HID, D_HID),
            _full_spec(D_HID, 2 * D_HID),
            _full_spec(1, D_HID),
            _full_spec(D_OUT, D_HID),
            _full_spec(1, D_OUT),
        ],
        out_specs=pl.BlockSpec((_BLK, D_OUT), lambda i: (i, 0)),
        out_shape=jax.ShapeDtypeStruct((N_NODES, D_OUT), jnp.float32),
    )(p1, inv, h0, wl1, bl1r, wr1, w1, b1r, w2, b2r)


def kernel(x, edge_index, Wl0, bl0, Wr0, Wl1, bl1, Wr1, W1, b1, W2, b2):
    edges = edge_index.astype(jnp.int32).reshape(2 * N_EDGES)
    z_hid = jnp.zeros((ROWS_PER_TILE, D_HID), jnp.float32)

    p0, cnt = _make_agg(True)(x, edges, z_hid)
    cnt_col = (cnt[0] + cnt[1]).reshape(N_PAD, 1)
    h0, inv = _dense0(p0, cnt_col, x, Wl0, bl0.reshape(1, -1), Wr0)
    p1 = _make_agg(False)(h0, edges, z_hid)[0]
    return _dense1(p1, inv, h0, Wl1, bl1.reshape(1, -1), Wr1,
                   W1, b1.reshape(1, -1), W2, b2.reshape(1, -1))
